# baseline (speedup 1.0000x reference)
import functools

import jax
import jax.numpy as jnp
from jax import lax
from jax.experimental import pallas as pl
from jax.experimental.pallas import tpu as pltpu
from jax.experimental.pallas import tpu_sc as plsc

N_SKILLS = 64
BATCH = 16384

_NC = 2
_NS = 16
_L = 16
_NW = _NC * _NS
_BW = BATCH // _NW

_mesh = plsc.VectorSubcoreMesh(core_axis_name="c", subcore_axis_name="s")


@functools.partial(
    pl.kernel,
    mesh=_mesh,
    out_type=jax.ShapeDtypeStruct((N_SKILLS, BATCH), jnp.float32),
    scratch_types=[
        pltpu.VMEM((_BW,), jnp.int32),
        pltpu.VMEM((N_SKILLS, _BW), jnp.float32),
    ],
)
def _sc_onehot(ids_hbm, out_hbm, idx_v, rows_v):
    wid = lax.axis_index("s") * _NC + lax.axis_index("c")
    base = wid * _BW
    pltpu.sync_copy(ids_hbm.at[pl.ds(base, _BW)], idx_v)

    one16 = jnp.ones((_L,), jnp.float32)
    zero16 = jnp.zeros((_L,), jnp.float32)

    def _fill_chunk(k, carry):
        ids16 = idx_v[pl.ds(k * _L, _L)]
        for j in range(N_SKILLS):
            rows_v[j, pl.ds(k * _L, _L)] = jnp.where(ids16 == j, one16, zero16)
        return carry

    lax.fori_loop(0, _BW // _L, _fill_chunk, 0)

    pltpu.sync_copy(rows_v, out_hbm.at[:, pl.ds(base, _BW)])


def kernel(task_ids):
    out = _sc_onehot(task_ids.astype(jnp.int32))
    return jnp.transpose(out, (1, 0))[:, None, :]


# speedup vs baseline: 7.4110x; 7.4110x over previous
import jax
import jax.numpy as jnp
from jax.experimental import pallas as pl

N_SKILLS = 64
BATCH = 16384


def _onehot_kernel(ids_ref, out_ref):
    jb = out_ref.shape[0]
    j0 = pl.program_id(0) * jb
    ids = ids_ref[:]
    iota_j = jax.lax.broadcasted_iota(jnp.int32, (jb, 128), 0) + j0
    for k in range(128):
        row = jnp.broadcast_to(ids[k : k + 1, :], (jb, 128))
        out_ref[:, k * 128 : (k + 1) * 128] = (row == iota_j).astype(jnp.float32)


def kernel(task_ids):
    ids2 = task_ids.reshape(128, 128).astype(jnp.int32)
    j_block = 32
    out = pl.pallas_call(
        _onehot_kernel,
        grid=(N_SKILLS // j_block,),
        in_specs=[pl.BlockSpec((128, 128), lambda i: (0, 0))],
        out_specs=pl.BlockSpec((j_block, BATCH), lambda i: (i, 0)),
        out_shape=jax.ShapeDtypeStruct((N_SKILLS, BATCH), jnp.float32),
    )(ids2)
    return jnp.transpose(out, (1, 0))[:, None, :]
